# baseline (device time: 59209 ns/iter reference)
import jax
import jax.numpy as jnp
from jax import lax
from jax.experimental import pallas as pl
from jax.experimental.pallas import tpu as pltpu

N_DEV = 4
SEND_ORDER = (1, 3, 2)
STEP_CHUNKS = (4, 2, 2)


def kernel(x, w_mat):
    m_per, k = x.shape
    _, n = w_mat.shape
    n_per = n // N_DEV
    x_rows = 256
    x_chunks = m_per // x_rows

    def body(x_hbm, w_hbm, out_ref,
             xf_ref, xbf_ref, wf_ref, wbf_ref, send_ref, recv_ref,
             xdma_sems, wdma_sems, send_sems, recv_sems):
        me = lax.axis_index("i")
        block_js = [(me + d) % N_DEV for d in SEND_ORDER] + [me]

        def x_dma(r, buf):
            return pltpu.make_async_copy(
                x_hbm.at[pl.ds(r * x_rows, x_rows), :],
                xf_ref.at[buf], xdma_sems.at[buf])

        def w_dma(s):
            return pltpu.make_async_copy(
                w_hbm.at[:, pl.ds(block_js[s] * n_per, n_per)],
                wf_ref.at[s % 2], wdma_sems.at[s % 2])

        def chunk_rdma(s, row0, nrows, r):
            return pltpu.make_async_remote_copy(
                src_ref=send_ref.at[s, pl.ds(row0, nrows)],
                dst_ref=recv_ref.at[s, pl.ds(row0, nrows)],
                send_sem=send_sems.at[s, r],
                recv_sem=recv_sems.at[s, r],
                device_id=(block_js[s],),
                device_id_type=pl.DeviceIdType.MESH)

        x_dma(0, 0).start()
        x_dma(1, 1).start()
        w_dma(0).start()
        w_dma(1).start()

        barrier_sem = pltpu.get_barrier_semaphore()
        for d in range(N_DEV):
            @pl.when(me != d)
            def _():
                pl.semaphore_signal(
                    barrier_sem, inc=1,
                    device_id=(d,), device_id_type=pl.DeviceIdType.MESH)
        pl.semaphore_wait(barrier_sem, N_DEV - 1)

        for r in range(x_chunks):
            x_dma(r, r % 2).wait()
            if r + 2 < x_chunks:
                x_dma(r + 2, r % 2).start()
            xbf_ref[pl.ds(r * x_rows, x_rows), :] = (
                xf_ref[r % 2].astype(jnp.bfloat16))

        w_dma(0).wait()
        wbf_ref[0] = wf_ref[0].astype(jnp.bfloat16)

        for s in range(3):
            w_dma(s + 1).wait()
            if s + 2 <= 3:
                w_dma(s + 2).start()
            nchunks = STEP_CHUNKS[s]
            rows_per = m_per // nchunks
            for h in range(nchunks):
                rows = pl.ds(h * rows_per, rows_per)
                y = jnp.dot(xbf_ref[rows, :], wbf_ref[s % 3],
                            preferred_element_type=jnp.float32)
                send_ref[s, rows, :] = (
                    jnp.maximum(y, 0.0).astype(jnp.bfloat16))
                chunk_rdma(s, h * rows_per, rows_per, h).start()
                if h == 0:
                    wbf_ref[(s + 1) % 3] = (
                        wf_ref[(s + 1) % 2].astype(jnp.bfloat16))

        y = jnp.dot(xbf_ref[...], wbf_ref[3 % 3],
                    preferred_element_type=jnp.float32)
        out_ref[pl.ds(me * m_per, m_per), :] = jnp.maximum(y, 0.0)

        for s, d in enumerate(SEND_ORDER):
            p = (me - d) % N_DEV
            nchunks = STEP_CHUNKS[s]
            rows_per = m_per // nchunks
            for r in range(nchunks):
                recv = pltpu.make_async_remote_copy(
                    src_ref=send_ref.at[s, pl.ds(r * rows_per, rows_per)],
                    dst_ref=recv_ref.at[s, pl.ds(r * rows_per, rows_per)],
                    send_sem=send_sems.at[s, r],
                    recv_sem=recv_sems.at[s, r],
                    device_id=(p,), device_id_type=pl.DeviceIdType.MESH)
                recv.wait_recv()
                out_ref[pl.ds(p * m_per + r * rows_per, rows_per), :] = (
                    recv_ref[s, pl.ds(r * rows_per, rows_per), :]
                    .astype(jnp.float32))

        for s in range(3):
            nchunks = STEP_CHUNKS[s]
            rows_per = m_per // nchunks
            for r in range(nchunks):
                send = pltpu.make_async_remote_copy(
                    src_ref=send_ref.at[s, pl.ds(r * rows_per, rows_per)],
                    dst_ref=recv_ref.at[s, pl.ds(r * rows_per, rows_per)],
                    send_sem=send_sems.at[s, r],
                    recv_sem=recv_sems.at[s, r],
                    device_id=(block_js[s],),
                    device_id_type=pl.DeviceIdType.MESH)
                send.wait_send()

    return pl.pallas_call(
        body,
        out_shape=jax.ShapeDtypeStruct((N_DEV * m_per, n_per), jnp.float32),
        in_specs=[
            pl.BlockSpec(memory_space=pl.ANY),
            pl.BlockSpec(memory_space=pl.ANY),
        ],
        out_specs=pl.BlockSpec(memory_space=pltpu.VMEM),
        scratch_shapes=[
            pltpu.VMEM((2, x_rows, k), jnp.float32),
            pltpu.VMEM((m_per, k), jnp.bfloat16),
            pltpu.VMEM((2, k, n_per), jnp.float32),
            pltpu.VMEM((3, k, n_per), jnp.bfloat16),
            pltpu.VMEM((3, m_per, n_per), jnp.bfloat16),
            pltpu.VMEM((3, m_per, n_per), jnp.bfloat16),
            pltpu.SemaphoreType.DMA((2,)),
            pltpu.SemaphoreType.DMA((2,)),
            pltpu.SemaphoreType.DMA((3, 4)),
            pltpu.SemaphoreType.DMA((3, 4)),
        ],
        compiler_params=pltpu.CompilerParams(
            collective_id=0,
            vmem_limit_bytes=63 * 1024 * 1024,
        ),
    )(x, w_mat)
